# baseline (speedup 1.0000x reference)
import jax
import jax.numpy as jnp
from jax.experimental import pallas as pl

SEQ_BLK = 256


def _add_kernel(x_ref, w_ref, o_ref):
    o_ref[...] = x_ref[...] + w_ref[...][:, None, :]


def kernel(x, weight):
    seq_len, batch, dim = x.shape
    grid = (seq_len // SEQ_BLK,)
    return pl.pallas_call(
        _add_kernel,
        grid=grid,
        in_specs=[
            pl.BlockSpec((SEQ_BLK, batch, dim), lambda i: (i, 0, 0)),
            pl.BlockSpec((SEQ_BLK, dim), lambda i: (i, 0)),
        ],
        out_specs=pl.BlockSpec((SEQ_BLK, batch, dim), lambda i: (i, 0, 0)),
        out_shape=jax.ShapeDtypeStruct((seq_len, batch, dim), x.dtype),
    )(x, weight[:seq_len])
